# Initial kernel scaffold; baseline (speedup 1.0000x reference)
import jax
import jax.numpy as jnp
from jax.experimental import pallas as pl


def kernel(features, edge_index, R):
    raise NotImplementedError("write your pallas kernel here")


# speedup vs baseline: 3.8270x; 3.8270x over previous
import functools
import math

import jax
import jax.numpy as jnp
from jax import lax
from jax.experimental import pallas as pl
from jax.experimental.pallas import tpu as pltpu
from jax.experimental.pallas import tpu_sc as plsc

ABS_MAX = 1000000.0
SCALE = 0.1
N = 10000
D = 128
E = 320000

NC = 2
NS = 16
NW = NC * NS

CHUNK = 128
CHUNKS_PER_TILE = -(-E // (NW * CHUNK))
EPT = CHUNKS_PER_TILE * CHUNK
E_PAD = EPT * NW
ROWS_PAD = 10240
RPT = ROWS_PAD // NS
ZROWS = 16

_mesh = plsc.VectorSubcoreMesh(
    core_axis_name="c", subcore_axis_name="s", num_cores=NC, num_subcores=NS)


@functools.partial(
    pl.kernel,
    out_type=jax.ShapeDtypeStruct((NC, ROWS_PAD, D), jnp.float32),
    mesh=_mesh,
    scratch_types=[
        pltpu.VMEM((CHUNK,), jnp.int32),
        pltpu.VMEM((CHUNK,), jnp.int32),
        pltpu.VMEM((CHUNK, D), jnp.float32),
        pltpu.VMEM((ZROWS, D), jnp.float32),
        pltpu.VMEM_SHARED((ROWS_PAD, D), jnp.float32),
        pltpu.SemaphoreType.DMA,
    ],
)
def _sc_scatter(feat_hbm, src_hbm, dst_hbm, out_hbm,
                src_v, dst_v, rows_v, zero_v, acc_sh, sem):
    c = lax.axis_index("c")
    s = lax.axis_index("s")
    wid = c * NS + s

    zeros16 = jnp.zeros((16,), jnp.float32)
    for r in range(ZROWS):
        for j in range(D // 16):
            zero_v[r, pl.ds(j * 16, 16)] = zeros16

    def zero_body(k, _):
        pltpu.sync_copy(zero_v, acc_sh.at[pl.ds(s * RPT + k * ZROWS, ZROWS)])
        return 0
    lax.fori_loop(0, RPT // ZROWS, zero_body, 0)

    plsc.subcore_barrier()

    def chunk_body(i, _):
        base = wid * EPT + i * CHUNK
        pltpu.sync_copy(src_hbm.at[pl.ds(base, CHUNK)], src_v)
        pltpu.sync_copy(dst_hbm.at[pl.ds(base, CHUNK)], dst_v)
        pltpu.async_copy(feat_hbm.at[src_v], rows_v, sem).wait()
        pltpu.sync_copy(rows_v, acc_sh.at[dst_v], add=True)
        return 0
    lax.fori_loop(0, CHUNKS_PER_TILE, chunk_body, 0)

    plsc.subcore_barrier()

    pltpu.sync_copy(acc_sh.at[pl.ds(s * RPT, RPT)],
                    out_hbm.at[c, pl.ds(s * RPT, RPT)])


def _tc_body(h_ref, f_ref, r_ref, o_ref):
    hsum = (h_ref[0] + h_ref[1]) * jnp.float32(SCALE)
    h = jnp.clip(hsum, -ABS_MAX, ABS_MAX)
    fe = jnp.clip(f_ref[...] * jnp.float32(SCALE), -ABS_MAX, ABS_MAX)
    feats = jnp.clip(h + fe, -ABS_MAX, ABS_MAX)
    o_ref[...] = lax.dot(feats, r_ref[...],
                         precision=lax.Precision.HIGHEST,
                         preferred_element_type=jnp.float32)


_BR = 1000

_tc_project = pl.pallas_call(
    _tc_body,
    grid=(N // _BR,),
    in_specs=[
        pl.BlockSpec((NC, _BR, D), lambda i: (0, i, 0)),
        pl.BlockSpec((_BR, D), lambda i: (i, 0)),
        pl.BlockSpec((D, D), lambda i: (0, 0)),
    ],
    out_specs=pl.BlockSpec((_BR, D), lambda i: (i, 0)),
    out_shape=jax.ShapeDtypeStruct((N, D), jnp.float32),
)


def kernel(features, edge_index, R):
    src = edge_index[0].astype(jnp.int32)
    dst = edge_index[1].astype(jnp.int32)
    npad = E_PAD - E
    src = jnp.concatenate([src, jnp.zeros((npad,), jnp.int32)])
    dst = jnp.concatenate([dst, jnp.full((npad,), N, jnp.int32)])
    h2 = _sc_scatter(features, src, dst)
    r_scaled = R * jnp.float32(1.0 / (math.sqrt(D) * math.sqrt(D)))
    return _tc_project(h2, features, r_scaled)
